# Initial kernel scaffold; baseline (speedup 1.0000x reference)
import jax
import jax.numpy as jnp
from jax.experimental import pallas as pl


def kernel(tags, categories, tag_table, cat_table):
    raise NotImplementedError("write your pallas kernel here")


# speedup vs baseline: 8.0583x; 8.0583x over previous
import functools

import jax
import jax.numpy as jnp
from jax import lax
from jax.experimental import pallas as pl
from jax.experimental.pallas import tpu as pltpu
from jax.experimental.pallas import tpu_sc as plsc

BATCH = 4096
EMBED_DIM = 64
TAG_LEN = 50
CAT_LEN = 20
LANES = 16
NGRP = EMBED_DIM // LANES

TAG_CHUNK_BAGS = 2
CAT_CHUNK_BAGS = 4


def _accumulate_chunk(rows_v, out_v, j, bags_per_chunk, bag_len):
    for bb in range(bags_per_chunk):
        base = bb * bag_len
        accs = tuple(rows_v[base, pl.ds(g * LANES, LANES)] for g in range(NGRP))

        def body(l, accs):
            return tuple(accs[g] + rows_v[base + l, pl.ds(g * LANES, LANES)]
                         for g in range(NGRP))

        accs = lax.fori_loop(1, bag_len, body, accs)
        bag = j * bags_per_chunk + bb
        for g in range(NGRP):
            out_v[bag, pl.ds(g * LANES, LANES)] = accs[g]


def _bag_phase(table_hbm, idx_v, rows_v, out_v, sems, nchunks, bags_per_chunk,
               bag_len):
    def start(j, s):
        pltpu.async_copy(table_hbm.at[idx_v.at[j]], rows_v.at[s], sems[s])

    def wait(j, s):
        pltpu.make_async_copy(table_hbm.at[idx_v.at[j]], rows_v.at[s],
                              sems[s]).wait()

    start(0, 0)

    def outer(jj, _):
        for s in range(2):
            j = jj * 2 + s
            wait(j, s)
            pl.when(j < nchunks - 1)(lambda: start(j + 1, 1 - s))
            _accumulate_chunk(rows_v.at[s], out_v, j, bags_per_chunk, bag_len)
        return _

    lax.fori_loop(0, nchunks // 2, outer, None)


def kernel(tags, categories, tag_table, cat_table):
    info = plsc.get_sparse_core_info()
    nw = info.num_cores * info.num_subcores
    bags_w = BATCH // nw

    tag_chunks = bags_w // TAG_CHUNK_BAGS
    cat_chunks = bags_w // CAT_CHUNK_BAGS
    tag_rows = TAG_CHUNK_BAGS * TAG_LEN
    cat_rows = CAT_CHUNK_BAGS * CAT_LEN

    tags_r = tags.reshape(nw, tag_chunks, tag_rows)
    cats_r = categories.reshape(nw, cat_chunks, cat_rows)

    mesh = plsc.VectorSubcoreMesh(core_axis_name="c", subcore_axis_name="s")
    out_sds = jax.ShapeDtypeStruct((BATCH, EMBED_DIM), jnp.float32)

    @functools.partial(
        pl.kernel,
        mesh=mesh,
        out_type=(out_sds, out_sds),
        compiler_params=pltpu.CompilerParams(use_tc_tiling_on_sc=False),
        scratch_types=[
            pltpu.VMEM((tag_chunks, tag_rows), jnp.int32),
            pltpu.VMEM((cat_chunks, cat_rows), jnp.int32),
            pltpu.VMEM((2, tag_rows, EMBED_DIM), jnp.float32),
            pltpu.VMEM((2, cat_rows, EMBED_DIM), jnp.float32),
            pltpu.VMEM((bags_w, EMBED_DIM), jnp.float32),
            pltpu.VMEM((bags_w, EMBED_DIM), jnp.float32),
            pltpu.SemaphoreType.DMA,
            pltpu.SemaphoreType.DMA,
        ],
    )
    def enc(tags_hbm, cats_hbm, ttab_hbm, ctab_hbm, out_t_hbm, out_c_hbm,
            tidx_v, cidx_v, trows_v, crows_v, tout_v, cout_v, sem0, sem1):
        wid = lax.axis_index("s") * info.num_cores + lax.axis_index("c")
        pltpu.sync_copy(tags_hbm.at[wid], tidx_v)
        pltpu.sync_copy(cats_hbm.at[wid], cidx_v)
        sems = (sem0, sem1)
        _bag_phase(ttab_hbm, tidx_v, trows_v, tout_v, sems, tag_chunks,
                   TAG_CHUNK_BAGS, TAG_LEN)
        _bag_phase(ctab_hbm, cidx_v, crows_v, cout_v, sems, cat_chunks,
                   CAT_CHUNK_BAGS, CAT_LEN)
        pltpu.sync_copy(tout_v, out_t_hbm.at[pl.ds(wid * bags_w, bags_w)])
        pltpu.sync_copy(cout_v, out_c_hbm.at[pl.ds(wid * bags_w, bags_w)])

    return enc(tags_r, cats_r, tag_table, cat_table)


# speedup vs baseline: 11.7076x; 1.4529x over previous
import functools

import jax
import jax.numpy as jnp
from jax import lax
from jax.experimental import pallas as pl
from jax.experimental.pallas import tpu as pltpu
from jax.experimental.pallas import tpu_sc as plsc

BATCH = 4096
EMBED_DIM = 64
TAG_LEN = 50
CAT_LEN = 20
LANES = 16
NGRP = EMBED_DIM // LANES
NSLOT = 4


def _make_phase(bag_len, chunk_bags, nw, bags_w):
    nchunks = bags_w // chunk_bags
    rows = chunk_bags * bag_len
    mesh = plsc.VectorSubcoreMesh(core_axis_name="c", subcore_axis_name="s")
    out_sds = jax.ShapeDtypeStruct((BATCH * EMBED_DIM,), jnp.float32)

    @functools.partial(
        pl.kernel,
        mesh=mesh,
        out_type=out_sds,
        compiler_params=pltpu.CompilerParams(use_tc_tiling_on_sc=False),
        scratch_types=[
            pltpu.VMEM((bags_w * bag_len,), jnp.int32),
            pltpu.VMEM((NSLOT, rows, EMBED_DIM), jnp.float32),
            pltpu.VMEM((bags_w * EMBED_DIM,), jnp.float32),
        ] + [pltpu.SemaphoreType.DMA] * NSLOT,
    )
    def enc(idx_hbm, tab_hbm, out_hbm, idx_v, rows_v, out_v, *sems):
        ncores = 2
        wid = lax.axis_index("s") * ncores + lax.axis_index("c")
        nidx = bags_w * bag_len
        pltpu.sync_copy(idx_hbm.at[pl.ds(wid * nidx, nidx)], idx_v)

        def start(j, s):
            idx = idx_v.at[pl.ds(j * rows, rows)]
            pltpu.async_copy(tab_hbm.at[idx], rows_v.at[s], sems[s])

        def wait(s):
            idx = idx_v.at[pl.ds(0, rows)]
            pltpu.make_async_copy(tab_hbm.at[idx], rows_v.at[s],
                                  sems[s]).wait()

        def accumulate(j, s):
            rv = rows_v.at[s]

            def bag_body(bb, _):
                base = bb * bag_len
                accs = [rv[base, pl.ds(g * LANES, LANES)] for g in range(NGRP)]
                for l in range(1, bag_len):
                    for g in range(NGRP):
                        accs[g] = accs[g] + rv[base + l,
                                               pl.ds(g * LANES, LANES)]
                out_base = (j * chunk_bags + bb) * EMBED_DIM
                for g in range(NGRP):
                    out_v[pl.ds(out_base + g * LANES, LANES)] = accs[g]
                return _

            lax.fori_loop(0, chunk_bags, bag_body, None)

        for s in range(NSLOT - 1):
            start(s, s)

        def outer(jj, _):
            for s in range(NSLOT):
                j = jj * NSLOT + s
                wait(s)
                nxt = j + NSLOT - 1
                pl.when(nxt < nchunks)(
                    lambda: start(nxt, (s + NSLOT - 1) % NSLOT))
                accumulate(j, s)
            return _

        lax.fori_loop(0, nchunks // NSLOT, outer, None)
        nout = bags_w * EMBED_DIM
        pltpu.sync_copy(out_v, out_hbm.at[pl.ds(wid * nout, nout)])

    return enc


def kernel(tags, categories, tag_table, cat_table):
    info = plsc.get_sparse_core_info()
    nw = info.num_cores * info.num_subcores
    bags_w = BATCH // nw

    ctab = cat_table.reshape(500, 128).reshape(1000, 64)
    ttab = tag_table.reshape(50000, 128).reshape(100000, 64)
    out_c = _make_phase(CAT_LEN, 4, nw, bags_w)(categories.reshape(-1), ctab)
    out_t = _make_phase(TAG_LEN, 4, nw, bags_w)(tags.reshape(-1), ttab)
    return (out_t.reshape(BATCH, EMBED_DIM), out_c.reshape(BATCH, EMBED_DIM))


# speedup vs baseline: 11.7373x; 1.0025x over previous
import functools

import jax
import jax.numpy as jnp
from jax import lax
from jax.experimental import pallas as pl
from jax.experimental.pallas import tpu as pltpu
from jax.experimental.pallas import tpu_sc as plsc

BATCH = 4096
EMBED_DIM = 64
TAG_LEN = 50
CAT_LEN = 20
LANES = 16
NGRP = EMBED_DIM // LANES
NSLOT = 4


def _make_phase(bag_len, chunk_bags, nw, bags_w):
    nchunks = bags_w // chunk_bags
    rows = chunk_bags * bag_len
    mesh = plsc.VectorSubcoreMesh(core_axis_name="c", subcore_axis_name="s")
    out_sds = jax.ShapeDtypeStruct((BATCH * EMBED_DIM,), jnp.float32)

    @functools.partial(
        pl.kernel,
        mesh=mesh,
        out_type=out_sds,
        compiler_params=pltpu.CompilerParams(use_tc_tiling_on_sc=False),
        scratch_types=[
            pltpu.VMEM((bags_w * bag_len,), jnp.int32),
            pltpu.VMEM((NSLOT, rows, EMBED_DIM), jnp.float32),
            pltpu.VMEM((bags_w * EMBED_DIM,), jnp.float32),
        ] + [pltpu.SemaphoreType.DMA] * NSLOT,
    )
    def enc(idx_hbm, tab_hbm, out_hbm, idx_v, rows_v, out_v, *sems):
        ncores = 2
        wid = lax.axis_index("s") * ncores + lax.axis_index("c")
        nidx = bags_w * bag_len
        pltpu.sync_copy(idx_hbm.at[pl.ds(wid * nidx, nidx)], idx_v)

        def start(j, s):
            idx = idx_v.at[pl.ds(j * rows, rows)]
            pltpu.async_copy(tab_hbm.at[idx], rows_v.at[s], sems[s])

        def wait(s):
            idx = idx_v.at[pl.ds(0, rows)]
            pltpu.make_async_copy(tab_hbm.at[idx], rows_v.at[s],
                                  sems[s]).wait()

        def accumulate(j, s):
            rv = rows_v.at[s]

            def bag_body(bb, _):
                base = bb * bag_len
                accs = [rv[base, pl.ds(g * LANES, LANES)] for g in range(NGRP)]
                for l in range(1, bag_len):
                    for g in range(NGRP):
                        accs[g] = accs[g] + rv[base + l,
                                               pl.ds(g * LANES, LANES)]
                out_base = (j * chunk_bags + bb) * EMBED_DIM
                for g in range(NGRP):
                    out_v[pl.ds(out_base + g * LANES, LANES)] = accs[g]
                return _

            lax.fori_loop(0, chunk_bags, bag_body, None)

        for s in range(NSLOT - 1):
            start(s, s)

        def outer(jj, _):
            for s in range(NSLOT):
                j = jj * NSLOT + s
                wait(s)
                nxt = j + NSLOT - 1
                pl.when(nxt < nchunks)(
                    lambda: start(nxt, (s + NSLOT - 1) % NSLOT))
                accumulate(j, s)
            return _

        lax.fori_loop(0, nchunks // NSLOT, outer, None)
        nout = bags_w * EMBED_DIM
        pltpu.sync_copy(out_v, out_hbm.at[pl.ds(wid * nout, nout)])

    return enc


def kernel(tags, categories, tag_table, cat_table):
    info = plsc.get_sparse_core_info()
    nw = info.num_cores * info.num_subcores
    bags_w = BATCH // nw

    ttab = jnp.pad(tag_table, ((0, 0), (0, EMBED_DIM))).reshape(-1, EMBED_DIM)
    ctab = jnp.pad(cat_table, ((0, 0), (0, EMBED_DIM))).reshape(-1, EMBED_DIM)
    out_c = _make_phase(CAT_LEN, 4, nw, bags_w)(
        categories.reshape(-1) * 2, ctab)
    out_t = _make_phase(TAG_LEN, 4, nw, bags_w)(tags.reshape(-1) * 2, ttab)
    return (out_t.reshape(BATCH, EMBED_DIM), out_c.reshape(BATCH, EMBED_DIM))


# speedup vs baseline: 12.2366x; 1.0425x over previous
import functools

import jax
import jax.numpy as jnp
from jax import lax
from jax.experimental import pallas as pl
from jax.experimental.pallas import tpu as pltpu
from jax.experimental.pallas import tpu_sc as plsc

BATCH = 4096
EMBED_DIM = 64
TAG_LEN = 50
CAT_LEN = 20
LANES = 16
NGRP = EMBED_DIM // LANES
NSLOT = 4


def _make_phase(bag_len, chunk_bags, nw, bags_w):
    nchunks = bags_w // chunk_bags
    rows = chunk_bags * bag_len
    mesh = plsc.VectorSubcoreMesh(core_axis_name="c", subcore_axis_name="s")
    out_sds = jax.ShapeDtypeStruct((BATCH * EMBED_DIM,), jnp.float32)

    @functools.partial(
        pl.kernel,
        mesh=mesh,
        out_type=out_sds,
        compiler_params=pltpu.CompilerParams(use_tc_tiling_on_sc=False),
        scratch_types=[
            pltpu.VMEM((bags_w * bag_len,), jnp.int32),
            pltpu.VMEM((NSLOT, rows, EMBED_DIM), jnp.float32),
            pltpu.VMEM((bags_w * EMBED_DIM,), jnp.float32),
        ] + [pltpu.SemaphoreType.DMA] * NSLOT,
    )
    def enc(idx_hbm, tab_hbm, out_hbm, idx_v, rows_v, out_v, *sems):
        ncores = 2
        wid = lax.axis_index("s") * ncores + lax.axis_index("c")
        nidx = bags_w * bag_len
        pltpu.sync_copy(idx_hbm.at[pl.ds(wid * nidx, nidx)], idx_v)

        def start(j, s):
            idx = idx_v.at[pl.ds(j * rows, rows)]
            pltpu.async_copy(tab_hbm.at[idx], rows_v.at[s], sems[s])

        def wait(s):
            idx = idx_v.at[pl.ds(0, rows)]
            pltpu.make_async_copy(tab_hbm.at[idx], rows_v.at[s],
                                  sems[s]).wait()

        def accumulate(j, s):
            rv = rows_v.at[s]

            unroll = 5

            def bag_body(bb, _):
                base = bb * bag_len
                accs = [rv[base, pl.ds(g * LANES, LANES)] for g in range(NGRP)]
                for l in range(1, unroll):
                    for g in range(NGRP):
                        accs[g] = accs[g] + rv[base + l,
                                               pl.ds(g * LANES, LANES)]

                def blk(t, accs):
                    row = base + t * unroll
                    for l in range(unroll):
                        accs = tuple(accs[g] + rv[row + l,
                                                  pl.ds(g * LANES, LANES)]
                                     for g in range(NGRP))
                    return accs

                accs = lax.fori_loop(1, bag_len // unroll, blk, tuple(accs))
                out_base = (j * chunk_bags + bb) * EMBED_DIM
                for g in range(NGRP):
                    out_v[pl.ds(out_base + g * LANES, LANES)] = accs[g]
                return _

            lax.fori_loop(0, chunk_bags, bag_body, None)

        for s in range(NSLOT - 1):
            start(s, s)

        def outer(jj, _):
            for s in range(NSLOT):
                j = jj * NSLOT + s
                wait(s)
                nxt = j + NSLOT - 1
                pl.when(nxt < nchunks)(
                    lambda: start(nxt, (s + NSLOT - 1) % NSLOT))
                accumulate(j, s)
            return _

        lax.fori_loop(0, nchunks // NSLOT, outer, None)
        nout = bags_w * EMBED_DIM
        pltpu.sync_copy(out_v, out_hbm.at[pl.ds(wid * nout, nout)])

    return enc


def kernel(tags, categories, tag_table, cat_table):
    info = plsc.get_sparse_core_info()
    nw = info.num_cores * info.num_subcores
    bags_w = BATCH // nw

    ttab = jnp.pad(tag_table, ((0, 0), (0, EMBED_DIM))).reshape(-1, EMBED_DIM)
    ctab = jnp.pad(cat_table, ((0, 0), (0, EMBED_DIM))).reshape(-1, EMBED_DIM)
    out_c = _make_phase(CAT_LEN, 4, nw, bags_w)(
        categories.reshape(-1) * 2, ctab)
    out_t = _make_phase(TAG_LEN, 4, nw, bags_w)(tags.reshape(-1) * 2, ttab)
    return (out_t.reshape(BATCH, EMBED_DIM), out_c.reshape(BATCH, EMBED_DIM))


# speedup vs baseline: 12.2498x; 1.0011x over previous
import functools

import jax
import jax.numpy as jnp
from jax import lax
from jax.experimental import pallas as pl
from jax.experimental.pallas import tpu as pltpu
from jax.experimental.pallas import tpu_sc as plsc

BATCH = 4096
EMBED_DIM = 64
TAG_LEN = 50
CAT_LEN = 20
LANES = 16
NGRP = EMBED_DIM // LANES
NSLOT = 4


def _make_phase(bag_len, chunk_bags, nw, bags_w):
    nchunks = bags_w // chunk_bags
    rows = chunk_bags * bag_len
    mesh = plsc.VectorSubcoreMesh(core_axis_name="c", subcore_axis_name="s")
    out_sds = jax.ShapeDtypeStruct((BATCH * EMBED_DIM,), jnp.float32)

    @functools.partial(
        pl.kernel,
        mesh=mesh,
        out_type=out_sds,
        compiler_params=pltpu.CompilerParams(use_tc_tiling_on_sc=False),
        scratch_types=[
            pltpu.VMEM((bags_w * bag_len,), jnp.int32),
            pltpu.VMEM((NSLOT, rows, EMBED_DIM), jnp.float32),
            pltpu.VMEM((bags_w * EMBED_DIM,), jnp.float32),
        ] + [pltpu.SemaphoreType.DMA] * NSLOT,
    )
    def enc(idx_hbm, tab_hbm, out_hbm, idx_v, rows_v, out_v, *sems):
        ncores = 2
        wid = lax.axis_index("s") * ncores + lax.axis_index("c")
        nidx = bags_w * bag_len
        pltpu.sync_copy(idx_hbm.at[pl.ds(wid * nidx, nidx)], idx_v)

        def start(j, s):
            idx = idx_v.at[pl.ds(j * rows, rows)]
            pltpu.async_copy(tab_hbm.at[idx], rows_v.at[s], sems[s])

        def wait(s):
            idx = idx_v.at[pl.ds(0, rows)]
            pltpu.make_async_copy(tab_hbm.at[idx], rows_v.at[s],
                                  sems[s]).wait()

        def accumulate(j, s):
            rv = rows_v.at[s]

            unroll = 5

            def bag_body(bb, _):
                base = bb * bag_len
                accs = [rv[base, pl.ds(g * LANES, LANES)] for g in range(NGRP)]
                for l in range(1, unroll):
                    for g in range(NGRP):
                        accs[g] = accs[g] + rv[base + l,
                                               pl.ds(g * LANES, LANES)]

                def blk(t, accs):
                    row = base + t * unroll
                    for l in range(unroll):
                        accs = tuple(accs[g] + rv[row + l,
                                                  pl.ds(g * LANES, LANES)]
                                     for g in range(NGRP))
                    return accs

                accs = lax.fori_loop(1, bag_len // unroll, blk, tuple(accs))
                out_base = (j * chunk_bags + bb) * EMBED_DIM
                for g in range(NGRP):
                    out_v[pl.ds(out_base + g * LANES, LANES)] = accs[g]
                return _

            lax.fori_loop(0, chunk_bags, bag_body, None)

        for s in range(NSLOT - 1):
            start(s, s)

        def outer(jj, _):
            for s in range(NSLOT):
                j = jj * NSLOT + s
                wait(s)
                nxt = j + NSLOT - 1
                pl.when(nxt < nchunks)(
                    lambda: start(nxt, (s + NSLOT - 1) % NSLOT))
                accumulate(j, s)
            return _

        lax.fori_loop(0, nchunks // NSLOT, outer, None)
        nout = bags_w * EMBED_DIM
        pltpu.sync_copy(out_v, out_hbm.at[pl.ds(wid * nout, nout)])

    return enc


def kernel(tags, categories, tag_table, cat_table):
    info = plsc.get_sparse_core_info()
    nw = info.num_cores * info.num_subcores
    bags_w = BATCH // nw

    ttab = jnp.pad(tag_table, ((0, 0), (0, EMBED_DIM))).reshape(-1, EMBED_DIM)
    ctab = jnp.pad(cat_table, ((0, 0), (0, EMBED_DIM))).reshape(-1, EMBED_DIM)
    out_c = _make_phase(CAT_LEN, 8, nw, bags_w)(
        categories.reshape(-1) * 2, ctab)
    out_t = _make_phase(TAG_LEN, 8, nw, bags_w)(tags.reshape(-1) * 2, ttab)
    return (out_t.reshape(BATCH, EMBED_DIM), out_c.reshape(BATCH, EMBED_DIM))
